# Initial kernel scaffold; baseline (speedup 1.0000x reference)
import jax
import jax.numpy as jnp
from jax.experimental import pallas as pl


def kernel(x, gate_weight):
    raise NotImplementedError("write your pallas kernel here")


# speedup vs baseline: 1.0686x; 1.0686x over previous
import jax
import jax.numpy as jnp
from jax.experimental import pallas as pl

NUM_EXPERTS = 64
TOP_K = 8
D_MODEL = 4096
N_TOKENS = 8192
BLOCK_R = 512


def _gate_kernel(x_ref, w_ref, ow_ref, oi_ref):
    x = x_ref[...]
    w = w_ref[...]
    logits = jax.lax.dot_general(
        x, w, (((1,), (1,)), ((), ())), preferred_element_type=jnp.float32
    )
    iota = jax.lax.broadcasted_iota(jnp.int32, logits.shape, 1)
    l = logits
    vals = []
    idxs = []
    for _ in range(TOP_K):
        m = jnp.max(l, axis=-1, keepdims=True)
        cand = jnp.where(l == m, iota, NUM_EXPERTS)
        idx = jnp.min(cand, axis=-1, keepdims=True)
        vals.append(m)
        idxs.append(idx)
        l = jnp.where(iota == idx, -jnp.inf, l)
    v = jnp.concatenate(vals, axis=1)
    i = jnp.concatenate(idxs, axis=1)
    e = jnp.exp(v - v[:, 0:1])
    ow_ref[...] = e / jnp.sum(e, axis=-1, keepdims=True)
    oi_ref[...] = i


def kernel(x, gate_weight):
    n, d = x.shape
    num_blocks = n // BLOCK_R
    ow, oi = pl.pallas_call(
        _gate_kernel,
        grid=(num_blocks,),
        in_specs=[
            pl.BlockSpec((BLOCK_R, d), lambda i: (i, 0)),
            pl.BlockSpec((NUM_EXPERTS, d), lambda i: (0, 0)),
        ],
        out_specs=[
            pl.BlockSpec((BLOCK_R, TOP_K), lambda i: (i, 0)),
            pl.BlockSpec((BLOCK_R, TOP_K), lambda i: (i, 0)),
        ],
        out_shape=[
            jax.ShapeDtypeStruct((n, TOP_K), jnp.float32),
            jax.ShapeDtypeStruct((n, TOP_K), jnp.int32),
        ],
    )(x, gate_weight)
    return ow, oi


# speedup vs baseline: 1.1278x; 1.0554x over previous
import jax
import jax.numpy as jnp
from jax.experimental import pallas as pl

NUM_EXPERTS = 64
TOP_K = 8
D_MODEL = 4096
N_TOKENS = 8192
BLOCK_R = 1024


def _gate_kernel(x_ref, w_ref, ow_ref, oi_ref):
    x = x_ref[...]
    w = w_ref[...]
    logits = jax.lax.dot_general(
        x, w, (((1,), (1,)), ((), ())), preferred_element_type=jnp.float32
    )
    iota = jax.lax.broadcasted_iota(jnp.int32, logits.shape, 1)
    l = logits
    vals = []
    idxs = []
    for _ in range(TOP_K):
        m = jnp.max(l, axis=-1, keepdims=True)
        cand = jnp.where(l == m, iota, NUM_EXPERTS)
        idx = jnp.min(cand, axis=-1, keepdims=True)
        vals.append(m)
        idxs.append(idx)
        l = jnp.where(iota == idx, -jnp.inf, l)
    v = jnp.concatenate(vals, axis=1)
    i = jnp.concatenate(idxs, axis=1)
    e = jnp.exp(v - v[:, 0:1])
    ow_ref[...] = e / jnp.sum(e, axis=-1, keepdims=True)
    oi_ref[...] = i


def kernel(x, gate_weight):
    n, d = x.shape
    num_blocks = n // BLOCK_R
    ow, oi = pl.pallas_call(
        _gate_kernel,
        grid=(num_blocks,),
        in_specs=[
            pl.BlockSpec((BLOCK_R, d), lambda i: (i, 0)),
            pl.BlockSpec((NUM_EXPERTS, d), lambda i: (0, 0)),
        ],
        out_specs=[
            pl.BlockSpec((BLOCK_R, TOP_K), lambda i: (i, 0)),
            pl.BlockSpec((BLOCK_R, TOP_K), lambda i: (i, 0)),
        ],
        out_shape=[
            jax.ShapeDtypeStruct((n, TOP_K), jnp.float32),
            jax.ShapeDtypeStruct((n, TOP_K), jnp.int32),
        ],
    )(x, gate_weight)
    return ow, oi


# speedup vs baseline: 1.2772x; 1.1325x over previous
import jax
import jax.numpy as jnp
from jax.experimental import pallas as pl

NUM_EXPERTS = 64
TOP_K = 8
BLOCK_R = 1024

_IDX_MASK = NUM_EXPERTS - 1


def _gate_kernel(x_ref, w_ref, ow_ref, oi_ref):
    logits = jax.lax.dot_general(
        x_ref[...], w_ref[...], (((1,), (1,)), ((), ())),
        preferred_element_type=jnp.float32,
    )
    bits = jax.lax.bitcast_convert_type(logits, jnp.int32)
    t = bits ^ jax.lax.shift_right_logical(
        jax.lax.shift_right_arithmetic(bits, 31), 1
    )
    iota = jax.lax.broadcasted_iota(jnp.int32, logits.shape, 1)
    key = (t & ~_IDX_MASK) | (_IDX_MASK - iota)
    kmin = jnp.int32(-(2**31))
    tops = []
    for _ in range(TOP_K):
        m = jnp.max(key, axis=-1, keepdims=True)
        tops.append(m)
        key = jnp.where(key == m, kmin, key)
    mk = jnp.concatenate(tops, axis=1)
    idx = _IDX_MASK - (mk & _IDX_MASK)
    t8 = mk & ~_IDX_MASK
    b8 = t8 ^ jax.lax.shift_right_logical(
        jax.lax.shift_right_arithmetic(t8, 31), 1
    )
    v = jax.lax.bitcast_convert_type(b8, jnp.float32)
    e = jnp.exp(v - v[:, 0:1])
    ow_ref[...] = e / jnp.sum(e, axis=-1, keepdims=True)
    oi_ref[...] = idx


def kernel(x, gate_weight):
    n, d = x.shape
    ow, oi = pl.pallas_call(
        _gate_kernel,
        grid=(n // BLOCK_R,),
        in_specs=[
            pl.BlockSpec((BLOCK_R, d), lambda i: (i, 0)),
            pl.BlockSpec((NUM_EXPERTS, d), lambda i: (0, 0)),
        ],
        out_specs=[
            pl.BlockSpec((BLOCK_R, TOP_K), lambda i: (i, 0)),
            pl.BlockSpec((BLOCK_R, TOP_K), lambda i: (i, 0)),
        ],
        out_shape=[
            jax.ShapeDtypeStruct((n, TOP_K), jnp.float32),
            jax.ShapeDtypeStruct((n, TOP_K), jnp.int32),
        ],
    )(x, gate_weight)
    return ow, oi


# speedup vs baseline: 1.5499x; 1.2135x over previous
import jax
import jax.numpy as jnp
from jax.experimental import pallas as pl

NUM_EXPERTS = 64
TOP_K = 8
BLOCK_R = 1024


def _gate_kernel(x_ref, w_ref, ow_ref, oi_ref):
    logits = jax.lax.dot_general(
        w_ref[...], x_ref[...], (((1,), (1,)), ((), ())),
        preferred_element_type=jnp.float32,
    )
    iota = jax.lax.broadcasted_iota(jnp.int32, logits.shape, 0)
    rev = (NUM_EXPERTS - 1) - iota
    l = logits
    vals = []
    idxs = []
    for _ in range(TOP_K):
        m = jnp.max(l, axis=0, keepdims=True)
        enc = jnp.where(l == m, rev, 0)
        idx = (NUM_EXPERTS - 1) - jnp.max(enc, axis=0, keepdims=True)
        vals.append(m)
        idxs.append(idx)
        l = jnp.where(iota == idx, -jnp.inf, l)
    v = jnp.concatenate(vals, axis=0)
    i = jnp.concatenate(idxs, axis=0)
    e = jnp.exp(v - v[0:1, :])
    w8 = e / jnp.sum(e, axis=0, keepdims=True)
    ow_ref[...] = w8.T
    oi_ref[...] = i.T


def kernel(x, gate_weight):
    n, d = x.shape
    ow, oi = pl.pallas_call(
        _gate_kernel,
        grid=(n // BLOCK_R,),
        in_specs=[
            pl.BlockSpec((BLOCK_R, d), lambda i: (i, 0)),
            pl.BlockSpec((NUM_EXPERTS, d), lambda i: (0, 0)),
        ],
        out_specs=[
            pl.BlockSpec((BLOCK_R, TOP_K), lambda i: (i, 0)),
            pl.BlockSpec((BLOCK_R, TOP_K), lambda i: (i, 0)),
        ],
        out_shape=[
            jax.ShapeDtypeStruct((n, TOP_K), jnp.float32),
            jax.ShapeDtypeStruct((n, TOP_K), jnp.int32),
        ],
    )(x, gate_weight)
    return ow, oi
